# baseline (speedup 1.0000x reference)
import functools

import jax
import jax.numpy as jnp
from jax import lax
from jax.experimental import pallas as pl
from jax.experimental.pallas import tpu as pltpu
from jax.experimental.pallas import tpu_sc as plsc

_VOCAB = 100000
_F = 8
_D = 16
_DM = 128
_NC, _NS, _LANES = 2, 16, 16
_NW = _NC * _NS

_CHUNK = 2048
_SLICES = 4


def _sc_gather_slice(y_flat, tables_flat, start, count):
    per_w = count // _NW
    nch = per_w // _CHUNK
    assert per_w % _CHUNK == 0

    mesh = plsc.VectorSubcoreMesh(
        core_axis_name="c", subcore_axis_name="s",
        num_cores=_NC, num_subcores=_NS)

    @functools.partial(
        pl.kernel,
        out_type=jax.ShapeDtypeStruct((count, _D), jnp.float32),
        mesh=mesh,
        scratch_types=[
            pltpu.VMEM((_CHUNK,), jnp.int32),
            pltpu.VMEM((_CHUNK,), jnp.int32),
            pltpu.VMEM((_CHUNK, _D), jnp.float32),
            pltpu.SemaphoreType.DMA,
        ],
        compiler_params=pltpu.CompilerParams(use_tc_tiling_on_sc=False),
    )
    def k(y_hbm, tab_hbm, out_hbm, y_v, idx_v, rows_v, sem):
        wid = lax.axis_index("s") * _NC + lax.axis_index("c")
        base = wid * per_w
        offvec = (lax.iota(jnp.int32, _LANES) % _F) * _VOCAB

        def chunk_body(ci, carry):
            off = base + ci * _CHUNK
            pltpu.sync_copy(y_hbm.at[pl.ds(start + off, _CHUNK)], y_v)

            def vec_body(j, c2):
                s = pl.multiple_of(j * _LANES, _LANES)
                idx_v[pl.ds(s, _LANES)] = y_v[pl.ds(s, _LANES)] + offvec
                return c2

            lax.fori_loop(0, _CHUNK // _LANES, vec_body, 0)
            pltpu.async_copy(tab_hbm.at[idx_v], rows_v, sem).wait()
            pltpu.sync_copy(rows_v, out_hbm.at[pl.ds(off, _CHUNK)])
            return carry

        lax.fori_loop(0, nch, chunk_body, 0)

    return k(y_flat, tables_flat)


def _tc_project(x, w, b2):
    m = x.shape[0]
    bm = 4096
    assert m % bm == 0

    def body(x_ref, w_ref, b_ref, o_ref):
        o_ref[...] = lax.dot_general(
            x_ref[...], w_ref[...], (((1,), (1,)), ((), ())),
            preferred_element_type=jnp.float32) + b_ref[...]

    return pl.pallas_call(
        body,
        grid=(m // bm,),
        in_specs=[
            pl.BlockSpec((bm, _DM), lambda i: (i, 0)),
            pl.BlockSpec((_DM, _DM), lambda i: (0, 0)),
            pl.BlockSpec((1, _DM), lambda i: (0, 0)),
        ],
        out_specs=pl.BlockSpec((bm, _DM), lambda i: (i, 0)),
        out_shape=jax.ShapeDtypeStruct((m, _DM), jnp.float32),
    )(x, w, b2)


def kernel(y, tables, W, b):
    bb, ll, ff = y.shape
    total = bb * ll * ff
    y_flat = y.reshape(-1)
    tab_flat = tables.reshape(_F * _VOCAB, _D)
    b2 = b.reshape(1, _DM)
    cnt = total // _SLICES
    outs = []
    for s in range(_SLICES):
        g = _sc_gather_slice(y_flat, tab_flat, s * cnt, cnt)
        x = g.reshape(cnt // _F, _F * _D)
        outs.append(_tc_project(x, W, b2))
    out = jnp.concatenate(outs, axis=0)
    return out.reshape(bb, ll, _DM)


# speedup vs baseline: 1.1676x; 1.1676x over previous
import functools

import jax
import jax.numpy as jnp
from jax import lax
from jax.experimental import pallas as pl
from jax.experimental.pallas import tpu as pltpu
from jax.experimental.pallas import tpu_sc as plsc

_VOCAB = 100000
_F = 8
_D = 16
_DM = 128
_NC, _NS, _LANES = 2, 16, 16
_NW = _NC * _NS

_CHUNK = 2048
_SLICES = 4
_BM = 4096


def _sc_mesh():
    return plsc.VectorSubcoreMesh(
        core_axis_name="c", subcore_axis_name="s",
        num_cores=_NC, num_subcores=_NS)


def _sc_prep(y, tables):
    bb, ll, ff = y.shape
    rows_w = bb // _NW
    yrc = 8
    trc = 2500

    @functools.partial(
        pl.kernel,
        out_type=[
            jax.ShapeDtypeStruct((bb, ll, ff), jnp.int32),
            jax.ShapeDtypeStruct((_F, _VOCAB, _D), jnp.float32),
        ],
        mesh=_sc_mesh(),
        scratch_types=[
            pltpu.VMEM((yrc, ll, ff), jnp.int32),
            pltpu.VMEM((trc, _D), jnp.float32),
        ],
        compiler_params=pltpu.CompilerParams(use_tc_tiling_on_sc=False),
    )
    def k(y_hbm, tab_hbm, yo_hbm, tabo_hbm, yv, tv):
        wid = lax.axis_index("s") * _NC + lax.axis_index("c")
        b0 = wid * rows_w

        def ybody(i, c):
            r = b0 + i * yrc
            pltpu.sync_copy(y_hbm.at[pl.ds(r, yrc)], yv)
            pltpu.sync_copy(yv, yo_hbm.at[pl.ds(r, yrc)])
            return c
        lax.fori_loop(0, rows_w // yrc, ybody, 0)

        f = wid // 4
        r0 = (wid % 4) * (_VOCAB // 4)

        def tbody(i, c):
            r = r0 + i * trc
            pltpu.sync_copy(tab_hbm.at[f, pl.ds(r, trc)], tv)
            pltpu.sync_copy(tv, tabo_hbm.at[f, pl.ds(r, trc)])
            return c
        lax.fori_loop(0, (_VOCAB // 4) // trc, tbody, 0)

    return k(y, tables)


def _sc_gather_slice(y_flat, tables_flat, start, count):
    per_w = count // _NW
    nch = per_w // _CHUNK
    assert per_w % _CHUNK == 0

    @functools.partial(
        pl.kernel,
        out_type=jax.ShapeDtypeStruct((count, _D), jnp.float32),
        mesh=_sc_mesh(),
        scratch_types=[
            pltpu.VMEM((_CHUNK,), jnp.int32),
            pltpu.VMEM((_CHUNK,), jnp.int32),
            pltpu.VMEM((_CHUNK, _D), jnp.float32),
            pltpu.SemaphoreType.DMA,
        ],
        compiler_params=pltpu.CompilerParams(use_tc_tiling_on_sc=False),
    )
    def k(y_hbm, tab_hbm, out_hbm, y_v, idx_v, rows_v, sem):
        wid = lax.axis_index("s") * _NC + lax.axis_index("c")
        base = wid * per_w
        offvec = (lax.iota(jnp.int32, _LANES) % _F) * _VOCAB

        def chunk_body(ci, carry):
            off = base + ci * _CHUNK
            pltpu.sync_copy(y_hbm.at[pl.ds(start + off, _CHUNK)], y_v)

            def vec_body(j, c2):
                s = pl.multiple_of(j * _LANES, _LANES)
                idx_v[pl.ds(s, _LANES)] = y_v[pl.ds(s, _LANES)] + offvec
                return c2

            lax.fori_loop(0, _CHUNK // _LANES, vec_body, 0)
            pltpu.async_copy(tab_hbm.at[idx_v], rows_v, sem).wait()
            pltpu.sync_copy(rows_v, out_hbm.at[pl.ds(off, _CHUNK)])
            return carry

        lax.fori_loop(0, nch, chunk_body, 0)

    return k(y_flat, tables_flat)


def _tc_project_slice(x, w, b2, acc, blk0, m_total):
    nblk = x.shape[0] // _BM
    assert x.shape[0] % _BM == 0

    def body(*refs):
        if len(refs) == 5:
            x_ref, w_ref, b_ref, _acc_ref, o_ref = refs
        else:
            x_ref, w_ref, b_ref, o_ref = refs
        o_ref[...] = lax.dot_general(
            x_ref[...], w_ref[...], (((1,), (1,)), ((), ())),
            preferred_element_type=jnp.float32) + b_ref[...]

    in_specs = [
        pl.BlockSpec((_BM, _DM), lambda i: (i, 0)),
        pl.BlockSpec((_DM, _DM), lambda i: (0, 0)),
        pl.BlockSpec((1, _DM), lambda i: (0, 0)),
    ]
    args = [x, w, b2]
    kwargs = {}
    if acc is not None:
        in_specs.append(pl.BlockSpec(memory_space=pl.ANY))
        args.append(acc)
        kwargs["input_output_aliases"] = {3: 0}

    return pl.pallas_call(
        body,
        grid=(nblk,),
        in_specs=in_specs,
        out_specs=pl.BlockSpec((_BM, _DM), lambda i: (i + blk0, 0)),
        out_shape=jax.ShapeDtypeStruct((m_total, _DM), jnp.float32),
        **kwargs,
    )(*args)


def kernel(y, tables, W, b):
    bb, ll, ff = y.shape
    total = bb * ll * ff
    m_total = bb * ll
    yd, tabd = _sc_prep(y, tables)
    y_flat = yd.reshape(-1)
    tab_flat = tabd.reshape(_F * _VOCAB, _D)
    b2 = b.reshape(1, _DM)
    cnt = total // _SLICES
    blk_per_slice = (cnt // _F) // _BM
    out = None
    for s in range(_SLICES):
        g = _sc_gather_slice(y_flat, tab_flat, s * cnt, cnt)
        x = g.reshape(cnt // _F, _F * _D)
        out = _tc_project_slice(x, W, b2, out, s * blk_per_slice, m_total)
    return out.reshape(bb, ll, _DM)


# speedup vs baseline: 1.2301x; 1.0536x over previous
import functools

import jax
import jax.numpy as jnp
from jax import lax
from jax.experimental import pallas as pl
from jax.experimental.pallas import tpu as pltpu
from jax.experimental.pallas import tpu_sc as plsc

_VOCAB = 100000
_F = 8
_D = 16
_DM = 128
_B, _L = 4096, 200
_NC, _NS, _LANES = 2, 16, 16
_NW = _NC * _NS

_SLICES = 4
_BS = _B // _SLICES
_BM = 4096


def _sc_mesh():
    return plsc.VectorSubcoreMesh(
        core_axis_name="c", subcore_axis_name="s",
        num_cores=_NC, num_subcores=_NS)


def _sc_table_transpose(tab_t):
    vq = _VOCAB // 4
    vc = 1000

    @functools.partial(
        pl.kernel,
        out_type=jax.ShapeDtypeStruct((_F * _VOCAB, _D), jnp.float32),
        mesh=_sc_mesh(),
        scratch_types=[pltpu.VMEM((vc, _D), jnp.float32)],
        compiler_params=pltpu.CompilerParams(use_tc_tiling_on_sc=False),
    )
    def k(t_hbm, o_hbm, stage):
        wid = lax.axis_index("s") * _NC + lax.axis_index("c")
        f = wid // 4
        v0 = (wid % 4) * vq

        def chunk(i, c):
            v = v0 + i * vc
            for d in range(_D):
                pltpu.sync_copy(t_hbm.at[f, d, pl.ds(v, vc)],
                                stage.at[pl.ds(0, vc), d])
            pltpu.sync_copy(stage, o_hbm.at[pl.ds(f * _VOCAB + v, vc)])
            return c
        lax.fori_loop(0, vq // vc, chunk, 0)

    return k(tab_t)


def _sc_gather_slice(y_t, tab2d, b0):
    pairs_w = (_L * _F) // _NW

    @functools.partial(
        pl.kernel,
        out_type=jax.ShapeDtypeStruct((_BS, _L * _F * _D), jnp.float32),
        mesh=_sc_mesh(),
        scratch_types=[
            pltpu.VMEM((_BS,), jnp.int32),
            pltpu.VMEM((_BS,), jnp.int32),
            pltpu.VMEM((_BS, _D), jnp.float32),
            pltpu.SemaphoreType.DMA,
        ],
        compiler_params=pltpu.CompilerParams(use_tc_tiling_on_sc=False),
    )
    def k(y_hbm, tab_hbm, out_hbm, y_v, idx_v, rows_v, sem):
        wid = lax.axis_index("s") * _NC + lax.axis_index("c")
        p0 = wid * pairs_w

        def pair_body(i, carry):
            p = p0 + i
            l = p // _F
            f = p % _F
            pltpu.sync_copy(y_hbm.at[l, f, pl.ds(b0, _BS)], y_v)
            off = f * _VOCAB

            def vec_body(j, c2):
                s = pl.multiple_of(j * _LANES, _LANES)
                idx_v[pl.ds(s, _LANES)] = y_v[pl.ds(s, _LANES)] + off
                return c2

            lax.fori_loop(0, _BS // _LANES, vec_body, 0)
            pltpu.async_copy(tab_hbm.at[idx_v], rows_v, sem).wait()
            pltpu.sync_copy(rows_v,
                            out_hbm.at[pl.ds(0, _BS), pl.ds(p * _D, _D)])
            return carry

        lax.fori_loop(0, pairs_w, pair_body, 0)

    return k(y_t, tab2d)


def _tc_project_slice(x, w, b2, acc, blk0, m_total):
    nblk = x.shape[0] // _BM
    assert x.shape[0] % _BM == 0

    def body(*refs):
        if len(refs) == 5:
            x_ref, w_ref, b_ref, _acc_ref, o_ref = refs
        else:
            x_ref, w_ref, b_ref, o_ref = refs
        o_ref[...] = lax.dot_general(
            x_ref[...], w_ref[...], (((1,), (1,)), ((), ())),
            preferred_element_type=jnp.float32) + b_ref[...]

    in_specs = [
        pl.BlockSpec((_BM, _DM), lambda i: (i, 0)),
        pl.BlockSpec((_DM, _DM), lambda i: (0, 0)),
        pl.BlockSpec((1, _DM), lambda i: (0, 0)),
    ]
    args = [x, w, b2]
    kwargs = {}
    if acc is not None:
        in_specs.append(pl.BlockSpec(memory_space=pl.ANY))
        args.append(acc)
        kwargs["input_output_aliases"] = {3: 0}

    return pl.pallas_call(
        body,
        grid=(nblk,),
        in_specs=in_specs,
        out_specs=pl.BlockSpec((_BM, _DM), lambda i: (i + blk0, 0)),
        out_shape=jax.ShapeDtypeStruct((m_total, _DM), jnp.float32),
        **kwargs,
    )(*args)


def kernel(y, tables, W, b):
    bb, ll, ff = y.shape
    m_total = bb * ll
    y_t = jnp.transpose(y, (1, 2, 0))
    tab2d = tables.reshape(_F * _VOCAB, _D)
    b2 = b.reshape(1, _DM)
    blk_per_slice = (_BS * ll) // _BM
    out = None
    for s in range(_SLICES):
        g = _sc_gather_slice(y_t, tab2d, s * _BS)
        x = g.reshape(_BS * ll, _F * _D)
        out = _tc_project_slice(x, W, b2, out, s * blk_per_slice, m_total)
    return out.reshape(bb, ll, _DM)


# speedup vs baseline: 1.6093x; 1.3082x over previous
import functools

import jax
import jax.numpy as jnp
from jax import lax
from jax.experimental import pallas as pl
from jax.experimental.pallas import tpu as pltpu
from jax.experimental.pallas import tpu_sc as plsc

_VOCAB = 100000
_F = 8
_D = 16
_DM = 128
_B, _L = 4096, 200
_LF = _L * _F
_NC, _NS, _LANES = 2, 16, 16
_NW = _NC * _NS

_SLICES = 4
_BS = _B // _SLICES
_CHUNK = 3200
_BM = 4096

_YBC = 16
_TVC = 1000


def _sc_mesh():
    return plsc.VectorSubcoreMesh(
        core_axis_name="c", subcore_axis_name="s",
        num_cores=_NC, num_subcores=_NS)


def _tc_build_idx(y2):
    bbk = 512

    def body(x_ref, o_ref):
        off = (lax.broadcasted_iota(jnp.int32, (bbk, _LF), 1) % _F) * _VOCAB
        o_ref[...] = x_ref[...].T + off

    return pl.pallas_call(
        body,
        grid=(_B // bbk,),
        in_specs=[pl.BlockSpec((_LF, bbk), lambda i: (0, i))],
        out_specs=pl.BlockSpec((bbk, _LF), lambda i: (i, 0)),
        out_shape=jax.ShapeDtypeStruct((_B, _LF), jnp.int32),
    )(y2)


def _sc_gather_slice(idx, tab2d, start, count):
    per_w = count // _NW
    nch = per_w // _CHUNK
    assert per_w % _CHUNK == 0

    @functools.partial(
        pl.kernel,
        out_type=jax.ShapeDtypeStruct((count, _D), jnp.float32),
        mesh=_sc_mesh(),
        scratch_types=[
            pltpu.VMEM((_CHUNK,), jnp.int32),
            pltpu.VMEM((_CHUNK, _D), jnp.float32),
            pltpu.SemaphoreType.DMA,
        ],
        compiler_params=pltpu.CompilerParams(use_tc_tiling_on_sc=False),
    )
    def k(idx_hbm, tab_hbm, out_hbm, idx_v, rows_v, sem):
        wid = lax.axis_index("s") * _NC + lax.axis_index("c")
        base = wid * per_w

        def chunk_body(ci, carry):
            off = base + ci * _CHUNK
            pltpu.sync_copy(idx_hbm.at[pl.ds(start + off, _CHUNK)], idx_v)
            pltpu.async_copy(tab_hbm.at[idx_v], rows_v, sem).wait()
            pltpu.sync_copy(rows_v, out_hbm.at[pl.ds(off, _CHUNK)])
            return carry

        lax.fori_loop(0, nch, chunk_body, 0)

    return k(idx, tab2d)


def _tc_project_slice(x, w, b2, acc, blk0, m_total):
    nblk = x.shape[0] // _BM
    assert x.shape[0] % _BM == 0

    def body(*refs):
        if len(refs) == 5:
            x_ref, w_ref, b_ref, _acc_ref, o_ref = refs
        else:
            x_ref, w_ref, b_ref, o_ref = refs
        o_ref[...] = lax.dot_general(
            x_ref[...], w_ref[...], (((1,), (1,)), ((), ())),
            preferred_element_type=jnp.float32) + b_ref[...]

    in_specs = [
        pl.BlockSpec((_BM, _DM), lambda i: (i, 0)),
        pl.BlockSpec((_DM, _DM), lambda i: (0, 0)),
        pl.BlockSpec((1, _DM), lambda i: (0, 0)),
    ]
    args = [x, w, b2]
    kwargs = {}
    if acc is not None:
        in_specs.append(pl.BlockSpec(memory_space=pl.ANY))
        args.append(acc)
        kwargs["input_output_aliases"] = {3: 0}

    return pl.pallas_call(
        body,
        grid=(nblk,),
        in_specs=in_specs,
        out_specs=pl.BlockSpec((_BM, _DM), lambda i: (i + blk0, 0)),
        out_shape=jax.ShapeDtypeStruct((m_total, _DM), jnp.float32),
        **kwargs,
    )(*args)


def kernel(y, tables, W, b):
    bb, ll, ff = y.shape
    m_total = bb * ll
    y2 = jnp.transpose(y, (1, 2, 0)).reshape(_LF, _B)
    idx = _tc_build_idx(y2).reshape(-1)
    tab2d = tables.reshape(_F * _VOCAB, _D)
    b2 = b.reshape(1, _DM)
    cnt = (bb * ll * ff) // _SLICES
    blk_per_slice = (_BS * ll) // _BM
    out = None
    for s in range(_SLICES):
        g = _sc_gather_slice(idx, tab2d, s * cnt, cnt)
        x = g.reshape(cnt // _F, _F * _D)
        out = _tc_project_slice(x, W, b2, out, s * blk_per_slice, m_total)
    return out.reshape(bb, ll, _DM)


# speedup vs baseline: 1.6113x; 1.0012x over previous
import functools

import jax
import jax.numpy as jnp
from jax import lax
from jax.experimental import pallas as pl
from jax.experimental.pallas import tpu as pltpu
from jax.experimental.pallas import tpu_sc as plsc

_VOCAB = 100000
_F = 8
_D = 16
_DM = 128
_B, _L = 4096, 200
_LF = _L * _F
_NC, _NS, _LANES = 2, 16, 16
_NW = _NC * _NS

_SLICES = 4
_BS = _B // _SLICES
_CHUNK = 3200
_BM = 4096

_VBK = 2048
_NVB = -(-_VOCAB // _VBK)
_VPAD = _NVB * _VBK


def _sc_mesh():
    return plsc.VectorSubcoreMesh(
        core_axis_name="c", subcore_axis_name="s",
        num_cores=_NC, num_subcores=_NS)


def _tc_build_idx(y2):
    bbk = 512

    def body(x_ref, o_ref):
        off = (lax.broadcasted_iota(jnp.int32, (bbk, _LF), 1) % _F) * _VOCAB
        o_ref[...] = (x_ref[...].T + off) * 8

    return pl.pallas_call(
        body,
        grid=(_B // bbk,),
        in_specs=[pl.BlockSpec((_LF, bbk), lambda i: (0, i))],
        out_specs=pl.BlockSpec((bbk, _LF), lambda i: (i, 0)),
        out_shape=jax.ShapeDtypeStruct((_B, _LF), jnp.int32),
    )(y2)


def _sc_gather_slice(idx, tab2d, start, count):
    per_w = count // _NW
    nch = per_w // _CHUNK
    assert per_w % _CHUNK == 0

    @functools.partial(
        pl.kernel,
        out_type=jax.ShapeDtypeStruct((count, _D), jnp.float32),
        mesh=_sc_mesh(),
        scratch_types=[
            pltpu.VMEM((_CHUNK,), jnp.int32),
            pltpu.VMEM((_CHUNK, _D), jnp.float32),
            pltpu.SemaphoreType.DMA,
        ],
        compiler_params=pltpu.CompilerParams(use_tc_tiling_on_sc=False),
    )
    def k(idx_hbm, tab_hbm, out_hbm, idx_v, rows_v, sem):
        wid = lax.axis_index("s") * _NC + lax.axis_index("c")
        base = wid * per_w

        def chunk_body(ci, carry):
            off = base + ci * _CHUNK
            pltpu.sync_copy(idx_hbm.at[pl.ds(start + off, _CHUNK)], idx_v)
            pltpu.async_copy(tab_hbm.at[idx_v], rows_v, sem).wait()
            pltpu.sync_copy(rows_v, out_hbm.at[pl.ds(off, _CHUNK)])
            return carry

        lax.fori_loop(0, nch, chunk_body, 0)

    return k(idx, tab2d)


def _tc_project_slice(x, w, b2, acc, blk0, m_total):
    nblk = x.shape[0] // _BM
    assert x.shape[0] % _BM == 0

    def body(*refs):
        if len(refs) == 5:
            x_ref, w_ref, b_ref, _acc_ref, o_ref = refs
        else:
            x_ref, w_ref, b_ref, o_ref = refs
        o_ref[...] = lax.dot_general(
            x_ref[...], w_ref[...], (((1,), (1,)), ((), ())),
            preferred_element_type=jnp.float32) + b_ref[...]

    in_specs = [
        pl.BlockSpec((_BM, _DM), lambda i: (i, 0)),
        pl.BlockSpec((_DM, _DM), lambda i: (0, 0)),
        pl.BlockSpec((1, _DM), lambda i: (0, 0)),
    ]
    args = [x, w, b2]
    kwargs = {}
    if acc is not None:
        in_specs.append(pl.BlockSpec(memory_space=pl.ANY))
        args.append(acc)
        kwargs["input_output_aliases"] = {3: 0}

    return pl.pallas_call(
        body,
        grid=(nblk,),
        in_specs=in_specs,
        out_specs=pl.BlockSpec((_BM, _DM), lambda i: (i + blk0, 0)),
        out_shape=jax.ShapeDtypeStruct((m_total, _DM), jnp.float32),
        **kwargs,
    )(*args)


def kernel(y, tables, W, b):
    bb, ll, ff = y.shape
    m_total = bb * ll
    y2 = jnp.transpose(y, (1, 2, 0)).reshape(_LF, _B)
    idx = _tc_build_idx(y2).reshape(-1)
    tabp = jnp.pad(tables, ((0, 0), (0, 0), (0, 128 - _D)))
    tab2d = tabp.reshape(_F * _VOCAB * 8, _D)
    b2 = b.reshape(1, _DM)
    cnt = (bb * ll * ff) // _SLICES
    blk_per_slice = (_BS * ll) // _BM
    out = None
    for s in range(_SLICES):
        g = _sc_gather_slice(idx, tab2d, s * cnt, cnt)
        x = g.reshape(cnt // _F, _F * _D)
        out = _tc_project_slice(x, W, b2, out, s * blk_per_slice, m_total)
    return out.reshape(bb, ll, _DM)


# speedup vs baseline: 1.6591x; 1.0297x over previous
import functools

import jax
import jax.numpy as jnp
from jax import lax
from jax.experimental import pallas as pl
from jax.experimental.pallas import tpu as pltpu
from jax.experimental.pallas import tpu_sc as plsc

_VOCAB = 100000
_F = 8
_D = 16
_DM = 128
_B, _L = 4096, 200
_LF = _L * _F
_NC, _NS, _LANES = 2, 16, 16
_NW = _NC * _NS

_SLICES = 4
_BS = _B // _SLICES
_CHUNK = 3200
_BM = 4096

_VBK = 2048
_NVB = -(-_VOCAB // _VBK)
_VPAD = _NVB * _VBK


def _sc_mesh():
    return plsc.VectorSubcoreMesh(
        core_axis_name="c", subcore_axis_name="s",
        num_cores=_NC, num_subcores=_NS)


def _tc_build_idx(y2):
    bbk = 512

    def body(x_ref, o_ref):
        off = (lax.broadcasted_iota(jnp.int32, (bbk, _LF), 1) % _F) * _VOCAB
        o_ref[...] = (x_ref[...].T + off) * 8

    return pl.pallas_call(
        body,
        grid=(_B // bbk,),
        in_specs=[pl.BlockSpec((_LF, bbk), lambda i: (0, i))],
        out_specs=pl.BlockSpec((bbk, _LF), lambda i: (i, 0)),
        out_shape=jax.ShapeDtypeStruct((_B, _LF), jnp.int32),
    )(y2)


def _sc_gather_slice(idx, tab2d, start, count):
    per_w = count // _NW
    nch = per_w // _CHUNK
    assert per_w % _CHUNK == 0

    @functools.partial(
        pl.kernel,
        out_type=jax.ShapeDtypeStruct((count, _D), jnp.float32),
        mesh=_sc_mesh(),
        scratch_types=[
            pltpu.VMEM((_CHUNK,), jnp.int32),
            pltpu.VMEM((_CHUNK,), jnp.int32),
            pltpu.VMEM((_CHUNK, _D), jnp.float32),
            pltpu.VMEM((_CHUNK, _D), jnp.float32),
            pltpu.SemaphoreType.DMA,
            pltpu.SemaphoreType.DMA,
        ],
        compiler_params=pltpu.CompilerParams(use_tc_tiling_on_sc=False),
    )
    def k(idx_hbm, tab_hbm, out_hbm, iv0, iv1, rv0, rv1, gsem, wsem):
        wid = lax.axis_index("s") * _NC + lax.axis_index("c")
        base = wid * per_w

        def pair_body(i, carry):
            c0 = base + (2 * i) * _CHUNK
            c1 = c0 + _CHUNK
            pltpu.sync_copy(idx_hbm.at[pl.ds(start + c0, _CHUNK)], iv0)
            g0 = pltpu.async_copy(tab_hbm.at[iv0], rv0, gsem)
            pltpu.sync_copy(idx_hbm.at[pl.ds(start + c1, _CHUNK)], iv1)
            g0.wait()
            w0 = pltpu.async_copy(rv0, out_hbm.at[pl.ds(c0, _CHUNK)], wsem)
            g1 = pltpu.async_copy(tab_hbm.at[iv1], rv1, gsem)
            g1.wait()
            w1 = pltpu.async_copy(rv1, out_hbm.at[pl.ds(c1, _CHUNK)], wsem)
            w0.wait()
            w1.wait()
            return carry

        lax.fori_loop(0, nch // 2, pair_body, 0)

    return k(idx, tab2d)


def _tc_project_slice(x, w, b2, acc, blk0, m_total):
    nblk = x.shape[0] // _BM
    assert x.shape[0] % _BM == 0

    def body(*refs):
        if len(refs) == 5:
            x_ref, w_ref, b_ref, _acc_ref, o_ref = refs
        else:
            x_ref, w_ref, b_ref, o_ref = refs
        o_ref[...] = lax.dot_general(
            x_ref[...], w_ref[...], (((1,), (1,)), ((), ())),
            preferred_element_type=jnp.float32) + b_ref[...]

    in_specs = [
        pl.BlockSpec((_BM, _DM), lambda i: (i, 0)),
        pl.BlockSpec((_DM, _DM), lambda i: (0, 0)),
        pl.BlockSpec((1, _DM), lambda i: (0, 0)),
    ]
    args = [x, w, b2]
    kwargs = {}
    if acc is not None:
        in_specs.append(pl.BlockSpec(memory_space=pl.ANY))
        args.append(acc)
        kwargs["input_output_aliases"] = {3: 0}

    return pl.pallas_call(
        body,
        grid=(nblk,),
        in_specs=in_specs,
        out_specs=pl.BlockSpec((_BM, _DM), lambda i: (i + blk0, 0)),
        out_shape=jax.ShapeDtypeStruct((m_total, _DM), jnp.float32),
        **kwargs,
    )(*args)


def kernel(y, tables, W, b):
    bb, ll, ff = y.shape
    m_total = bb * ll
    y2 = jnp.transpose(y, (1, 2, 0)).reshape(_LF, _B)
    idx = _tc_build_idx(y2).reshape(-1)
    tabp = jnp.pad(tables, ((0, 0), (0, 0), (0, 128 - _D)))
    tab2d = tabp.reshape(_F * _VOCAB * 8, _D)
    b2 = b.reshape(1, _DM)
    cnt = (bb * ll * ff) // _SLICES
    blk_per_slice = (_BS * ll) // _BM
    out = None
    for s in range(_SLICES):
        g = _sc_gather_slice(idx, tab2d, s * cnt, cnt)
        x = g.reshape(cnt // _F, _F * _D)
        out = _tc_project_slice(x, W, b2, out, s * blk_per_slice, m_total)
    return out.reshape(bb, ll, _DM)


# speedup vs baseline: 1.6665x; 1.0045x over previous
import functools

import jax
import jax.numpy as jnp
from jax import lax
from jax.experimental import pallas as pl
from jax.experimental.pallas import tpu as pltpu
from jax.experimental.pallas import tpu_sc as plsc

_VOCAB = 100000
_F = 8
_D = 16
_DM = 128
_B, _L = 4096, 200
_LF = _L * _F
_NC, _NS, _LANES = 2, 16, 16
_NW = _NC * _NS

_SLICES = 8
_BS = _B // _SLICES
_CHUNK = 3200
_BM = 4096

_VBK = 2048
_NVB = -(-_VOCAB // _VBK)
_VPAD = _NVB * _VBK


def _sc_mesh():
    return plsc.VectorSubcoreMesh(
        core_axis_name="c", subcore_axis_name="s",
        num_cores=_NC, num_subcores=_NS)


def _tc_build_idx(y2):
    bbk = 512

    def body(x_ref, o_ref):
        off = (lax.broadcasted_iota(jnp.int32, (bbk, _LF), 1) % _F) * _VOCAB
        o_ref[...] = (x_ref[...].T + off) * 8

    return pl.pallas_call(
        body,
        grid=(_B // bbk,),
        in_specs=[pl.BlockSpec((_LF, bbk), lambda i: (0, i))],
        out_specs=pl.BlockSpec((bbk, _LF), lambda i: (i, 0)),
        out_shape=jax.ShapeDtypeStruct((_B, _LF), jnp.int32),
    )(y2)


def _sc_gather_slice(idx, tab2d, start, count):
    per_w = count // _NW
    nch = per_w // _CHUNK
    assert per_w % _CHUNK == 0

    @functools.partial(
        pl.kernel,
        out_type=jax.ShapeDtypeStruct((count, _D), jnp.float32),
        mesh=_sc_mesh(),
        scratch_types=[
            pltpu.VMEM((_CHUNK,), jnp.int32),
            pltpu.VMEM((_CHUNK,), jnp.int32),
            pltpu.VMEM((_CHUNK, _D), jnp.float32),
            pltpu.VMEM((_CHUNK, _D), jnp.float32),
            pltpu.SemaphoreType.DMA,
            pltpu.SemaphoreType.DMA,
        ],
        compiler_params=pltpu.CompilerParams(use_tc_tiling_on_sc=False),
    )
    def k(idx_hbm, tab_hbm, out_hbm, iv0, iv1, rv0, rv1, gsem, wsem):
        wid = lax.axis_index("s") * _NC + lax.axis_index("c")
        base = wid * per_w

        def pair_body(i, carry):
            c0 = base + (2 * i) * _CHUNK
            c1 = c0 + _CHUNK
            pltpu.sync_copy(idx_hbm.at[pl.ds(start + c0, _CHUNK)], iv0)
            g0 = pltpu.async_copy(tab_hbm.at[iv0], rv0, gsem)
            pltpu.sync_copy(idx_hbm.at[pl.ds(start + c1, _CHUNK)], iv1)
            g0.wait()
            w0 = pltpu.async_copy(rv0, out_hbm.at[pl.ds(c0, _CHUNK)], wsem)
            g1 = pltpu.async_copy(tab_hbm.at[iv1], rv1, gsem)
            g1.wait()
            w1 = pltpu.async_copy(rv1, out_hbm.at[pl.ds(c1, _CHUNK)], wsem)
            w0.wait()
            w1.wait()
            return carry

        lax.fori_loop(0, nch // 2, pair_body, 0)

    return k(idx, tab2d)


def _tc_project_slice(x, w, b2, acc, blk0, m_total):
    nblk = x.shape[0] // _BM
    assert x.shape[0] % _BM == 0

    def body(*refs):
        if len(refs) == 5:
            x_ref, w_ref, b_ref, _acc_ref, o_ref = refs
        else:
            x_ref, w_ref, b_ref, o_ref = refs
        o_ref[...] = lax.dot_general(
            x_ref[...], w_ref[...], (((1,), (1,)), ((), ())),
            preferred_element_type=jnp.float32) + b_ref[...]

    in_specs = [
        pl.BlockSpec((_BM, _DM), lambda i: (i, 0)),
        pl.BlockSpec((_DM, _DM), lambda i: (0, 0)),
        pl.BlockSpec((1, _DM), lambda i: (0, 0)),
    ]
    args = [x, w, b2]
    kwargs = {}
    if acc is not None:
        in_specs.append(pl.BlockSpec(memory_space=pl.ANY))
        args.append(acc)
        kwargs["input_output_aliases"] = {3: 0}

    return pl.pallas_call(
        body,
        grid=(nblk,),
        in_specs=in_specs,
        out_specs=pl.BlockSpec((_BM, _DM), lambda i: (i + blk0, 0)),
        out_shape=jax.ShapeDtypeStruct((m_total, _DM), jnp.float32),
        **kwargs,
    )(*args)


def kernel(y, tables, W, b):
    bb, ll, ff = y.shape
    m_total = bb * ll
    y2 = jnp.transpose(y, (1, 2, 0)).reshape(_LF, _B)
    idx = _tc_build_idx(y2).reshape(-1)
    tabp = jnp.pad(tables, ((0, 0), (0, 0), (0, 128 - _D)))
    tab2d = tabp.reshape(_F * _VOCAB * 8, _D)
    b2 = b.reshape(1, _DM)
    cnt = (bb * ll * ff) // _SLICES
    blk_per_slice = (_BS * ll) // _BM
    out = None
    for s in range(_SLICES):
        g = _sc_gather_slice(idx, tab2d, s * cnt, cnt)
        x = g.reshape(cnt // _F, _F * _D)
        out = _tc_project_slice(x, W, b2, out, s * blk_per_slice, m_total)
    return out.reshape(bb, ll, _DM)
